# Initial kernel scaffold; baseline (speedup 1.0000x reference)
import jax
import jax.numpy as jnp
from jax.experimental import pallas as pl


def kernel(x, edge_index, pos_edge_index, neg_edge_index, temp, W0, b0, W1, b1, Wm, bm, Wl, bl, Wq, bq, Wd1, bd1, WdX, bdX, Wa, ba):
    raise NotImplementedError("write your pallas kernel here")


# speedup vs baseline: 1.0086x; 1.0086x over previous
import functools

import jax
import jax.numpy as jnp
from jax.experimental import pallas as pl
from jax.experimental.pallas import tpu as pltpu

_N = 10000
_E = 320000
_EP = 100000
_CAT = 8
_H2 = 256
_EPS = 1e-07
_N_PAD = 10240
_R_BLK = 2000


def _full_spec(ndim):
    return pl.BlockSpec(index_map=lambda i: (0,) * ndim)


def _prep_body(dego_ref, degi_ref, x_ref, xs_ref, nout_ref, nin_ref):
    dego = dego_ref[0] + dego_ref[1]
    degi = degi_ref[0] + degi_ref[1]
    nout = jnp.where(dego > 0, jax.lax.rsqrt(jnp.maximum(dego, 1e-30)), 0.0)
    nin = jnp.where(degi > 0, jax.lax.rsqrt(jnp.maximum(degi, 1e-30)), 0.0)
    xs_ref[...] = x_ref[...] * nout
    nout_ref[...] = nout
    nin_ref[...] = nin


def _prep(deg_out_p, deg_in_p, x_pad):
    return pl.pallas_call(
        _prep_body,
        out_shape=[
            jax.ShapeDtypeStruct((_N_PAD, 128), jnp.float32),
            jax.ShapeDtypeStruct((_N_PAD, 1), jnp.float32),
            jax.ShapeDtypeStruct((_N_PAD, 1), jnp.float32),
        ],
    )(deg_out_p, deg_in_p, x_pad)


def _layer0_body(agg_ref, nin_ref, nout_ref, W_ref, b_ref, out_ref):
    agg = agg_ref[0] + agg_ref[1]
    h = jnp.dot(agg * nin_ref[...], W_ref[...],
                preferred_element_type=jnp.float32) + b_ref[...]
    out_ref[...] = jnp.maximum(h, 0.0) * nout_ref[...]


def _layer0(agg_p, nin, nout, W, b):
    return pl.pallas_call(
        _layer0_body,
        out_shape=jax.ShapeDtypeStruct((_N_PAD, 128), jnp.float32),
    )(agg_p, nin, nout, W, b.reshape(1, 128))


def _layer1_body(agg_ref, nin_ref, W_ref, b_ref, out_ref):
    agg = agg_ref[0] + agg_ref[1]
    out_ref[...] = jnp.dot(agg * nin_ref[...], W_ref[...],
                           preferred_element_type=jnp.float32) + b_ref[...]


def _layer1(agg_p, nin, W, b):
    return pl.pallas_call(
        _layer1_body,
        out_shape=jax.ShapeDtypeStruct((_N_PAD, 128), jnp.float32),
    )(agg_p, nin, W, b.reshape(1, 128))


def _branch_body(temp_ref, hu_ref, hv_ref, noise_ref, unif_ref,
                 Wm_ref, bm_ref, Wl_ref, bl_ref, Wq_ref, bq_ref,
                 Wd1_ref, bd1_ref, WdX_ref, bdX_ref, Wa_ref, ba_ref,
                 A_ref, X_ref, mean_ref, logstd_ref, q_ref):
    npemb = hu_ref[...] + hv_ref[...]
    mean = jnp.dot(npemb, Wm_ref[...],
                   preferred_element_type=jnp.float32) + bm_ref[...]
    logstd = jnp.dot(npemb, Wl_ref[...],
                     preferred_element_type=jnp.float32) + bl_ref[...]
    q = jnp.dot(npemb, Wq_ref[...],
                preferred_element_type=jnp.float32) + bq_ref[...]
    Nz = noise_ref[...] * jnp.exp(logstd) + mean
    g = -jnp.log(-jnp.log(unif_ref[...] + _EPS) + _EPS)
    t = temp_ref[0, 0]
    Z = jax.nn.softmax((q + g) / t, axis=-1)
    M = jnp.zeros((_R_BLK, 32), jnp.float32)
    for c in range(_CAT):
        M = M + Z[:, c:c + 1] * Nz[:, c * 32:(c + 1) * 32]
    A_ref[...] = jax.nn.softmax(
        jnp.dot(M, Wa_ref[...], preferred_element_type=jnp.float32)
        + ba_ref[...], axis=-1)
    Xh = jnp.maximum(
        jnp.dot(M, Wd1_ref[...], preferred_element_type=jnp.float32)
        + bd1_ref[...], 0.0)
    X_ref[...] = jnp.maximum(
        jnp.dot(Xh, WdX_ref[...], preferred_element_type=jnp.float32)
        + bdX_ref[...], 0.0)
    mean_ref[...] = mean
    logstd_ref[...] = logstd
    q_ref[...] = q


def _branch(temp_arr, hu, hv, noise, unif,
            Wm, bm, Wl, bl, Wq, bq, Wd1, bd1, WdX, bdX, Wa, ba):
    nblk = _EP // _R_BLK
    row = lambda i: (i, 0)
    return pl.pallas_call(
        _branch_body,
        grid=(nblk,),
        in_specs=[
            _full_spec(2),
            pl.BlockSpec((_R_BLK, 128), row),
            pl.BlockSpec((_R_BLK, 128), row),
            pl.BlockSpec((_R_BLK, _H2), row),
            pl.BlockSpec((_R_BLK, _CAT), row),
            _full_spec(2), _full_spec(2),
            _full_spec(2), _full_spec(2),
            _full_spec(2), _full_spec(2),
            _full_spec(2), _full_spec(2),
            _full_spec(2), _full_spec(2),
            _full_spec(2), _full_spec(2),
        ],
        out_specs=[
            pl.BlockSpec((_R_BLK, _CAT), row),
            pl.BlockSpec((_R_BLK, 128), row),
            pl.BlockSpec((_R_BLK, _H2), row),
            pl.BlockSpec((_R_BLK, _H2), row),
            pl.BlockSpec((_R_BLK, _CAT), row),
        ],
        out_shape=[
            jax.ShapeDtypeStruct((_EP, _CAT), jnp.float32),
            jax.ShapeDtypeStruct((_EP, 128), jnp.float32),
            jax.ShapeDtypeStruct((_EP, _H2), jnp.float32),
            jax.ShapeDtypeStruct((_EP, _H2), jnp.float32),
            jax.ShapeDtypeStruct((_EP, _CAT), jnp.float32),
        ],
    )(temp_arr, hu, hv, noise, unif,
      Wm, bm.reshape(1, _H2), Wl, bl.reshape(1, _H2),
      Wq, bq.reshape(1, _CAT), Wd1, bd1.reshape(1, 64),
      WdX, bdX.reshape(1, 128), Wa, ba.reshape(1, _CAT))


def _pad_deg(deg):
    out = jnp.zeros((2, _N_PAD, 1), jnp.float32)
    return out.at[0, :_N, 0].set(deg.astype(jnp.float32))


def _segsum(table_pad, src, dst):
    agg = jax.ops.segment_sum(table_pad[src], dst, num_segments=_N)
    out = jnp.zeros((2, _N_PAD, 128), jnp.float32)
    return out.at[0, :_N].set(agg)


def kernel(x, edge_index, pos_edge_index, neg_edge_index, temp,
           W0, b0, W1, b1, Wm, bm, Wl, bl, Wq, bq,
           Wd1, bd1, WdX, bdX, Wa, ba):
    src, dst = edge_index[0], edge_index[1]
    deg_out_p = _pad_deg(jnp.bincount(src, length=_N))
    deg_in_p = _pad_deg(jnp.bincount(dst, length=_N))
    x_pad = jnp.pad(x, ((0, _N_PAD - _N), (0, 0)))

    xs, nout, nin = _prep(deg_out_p, deg_in_p, x_pad)
    agg0_p = _segsum(xs, src, dst)
    h0s = _layer0(agg0_p, nin, nout, W0, b0)
    agg1_p = _segsum(h0s, src, dst)
    h = _layer1(agg1_p, nin, W1, b1)

    temp_arr = jnp.asarray(temp, jnp.float32).reshape(1, 1)
    noise_p = jax.random.normal(jax.random.key(42), (_EP, _H2), jnp.float32)
    unif_p = jax.random.uniform(jax.random.key(43), (_EP, _CAT), jnp.float32)
    noise_n = jax.random.normal(jax.random.key(44), (_EP, _H2), jnp.float32)
    unif_n = jax.random.uniform(jax.random.key(45), (_EP, _CAT), jnp.float32)

    hu_p, hv_p = h[pos_edge_index[0]], h[pos_edge_index[1]]
    hu_n, hv_n = h[neg_edge_index[0]], h[neg_edge_index[1]]

    posA, posX, pos_mean, pos_logstd, posq = _branch(
        temp_arr, hu_p, hv_p, noise_p, unif_p,
        Wm, bm, Wl, bl, Wq, bq, Wd1, bd1, WdX, bdX, Wa, ba)
    negA, negX, neg_mean, neg_logstd, negq = _branch(
        temp_arr, hu_n, hv_n, noise_n, unif_n,
        Wm, bm, Wl, bl, Wq, bq, Wd1, bd1, WdX, bdX, Wa, ba)

    return (posA, negA, posX, negX, pos_mean, neg_mean,
            pos_logstd, neg_logstd, posq, negq)


# speedup vs baseline: 1.5798x; 1.5663x over previous
import functools

import jax
import jax.numpy as jnp
from jax import lax
from jax.experimental import pallas as pl
from jax.experimental.pallas import tpu as pltpu
from jax.experimental.pallas import tpu_sc as plsc

_N = 10000
_E = 320000
_EP = 100000
_CAT = 8
_H2 = 256
_EPS = 1e-07
_N_PAD = 10240
_R_BLK = 2000

_NC = 2
_NS = 16
_NW = _NC * _NS
_E_PAD = 327680
_EROWS = _E_PAD // 128
_RPT = _EROWS // _NW
_P_PAD = 106496
_PROWS = 4 * _P_PAD // 128
_RPT_P = _PROWS // _NW
_NZ = _N_PAD // _NS


def _sc_mesh():
    return plsc.VectorSubcoreMesh(core_axis_name="c", subcore_axis_name="s")


@functools.lru_cache(maxsize=None)
def _sc_degrees_k():
    @functools.partial(
        pl.kernel,
        out_type=jax.ShapeDtypeStruct((2, 2, _N_PAD), jnp.float32),
        mesh=_sc_mesh(),
        scratch_types=[
            pltpu.VMEM((2 * _RPT, 128), jnp.int32),
            pltpu.VMEM((128,), jnp.float32),
            pltpu.VMEM((_NZ,), jnp.float32),
            pltpu.VMEM_SHARED((_N_PAD,), jnp.float32),
            pltpu.VMEM_SHARED((_N_PAD,), jnp.float32),
            pltpu.SemaphoreType.DMA((4,)),
        ],
    )
    def _sc_degrees(edges_hbm, out_hbm, idx_v, ones_v, zb_v, acc0, acc1,
                    sems):
        c = lax.axis_index("c")
        s = lax.axis_index("s")
        wid = s * _NC + c
        for i in range(8):
            ones_v[pl.ds(16 * i, 16)] = jnp.ones((16,), jnp.float32)
        for i in range(_NZ // 16):
            zb_v[pl.ds(16 * i, 16)] = jnp.zeros((16,), jnp.float32)
        pltpu.sync_copy(zb_v, acc0.at[pl.ds(s * _NZ, _NZ)])
        pltpu.sync_copy(zb_v, acc1.at[pl.ds(s * _NZ, _NZ)])
        for t in range(2):
            pltpu.sync_copy(edges_hbm.at[t, pl.ds(wid * _RPT, _RPT)],
                            idx_v.at[pl.ds(t * _RPT, _RPT)])
        plsc.subcore_barrier()
        for t, acc in ((0, acc0), (1, acc1)):
            def body(g, _, t=t, acc=acc):
                descs = []
                for k in range(4):
                    j = t * _RPT + g * 4 + k
                    descs.append(pltpu.async_copy(
                        ones_v, acc.at[idx_v.at[j]], sems.at[k], add=True))
                for d in descs:
                    d.wait()
                return ()
            lax.fori_loop(0, _RPT // 4, body, ())
        plsc.subcore_barrier()
        pltpu.sync_copy(acc0.at[pl.ds(s * _NZ, _NZ)],
                        out_hbm.at[0, c, pl.ds(s * _NZ, _NZ)])
        pltpu.sync_copy(acc1.at[pl.ds(s * _NZ, _NZ)],
                        out_hbm.at[1, c, pl.ds(s * _NZ, _NZ)])

    return _sc_degrees


_RPC = _EROWS // _NS
_NQ = _N_PAD // 4
_A_PAD = _NQ + 128
_NZ2 = _A_PAD // _NS


@functools.lru_cache(maxsize=None)
def _sc_segsum_k():
    @functools.partial(
        pl.kernel,
        out_type=jax.ShapeDtypeStruct((2, 2, _A_PAD, 128), jnp.float32),
        mesh=_sc_mesh(),
        scratch_types=[
            pltpu.VMEM((2 * _RPC, 128), jnp.int32),
            pltpu.VMEM((4, 128, 128), jnp.float32),
            pltpu.VMEM((21, 128), jnp.float32),
            pltpu.VMEM_SHARED((_A_PAD, 128), jnp.float32),
            pltpu.SemaphoreType.DMA((4,)),
            pltpu.SemaphoreType.DMA((4,)),
            pltpu.SemaphoreType.DMA((8,)),
        ],
    )
    def _sc_segsum(table_hbm, gidx_hbm, sidx_hbm, out_hbm,
                   idx_v, buf_v, zb_v, acc, gsems, ssems, zsems):
        c = lax.axis_index("c")
        s = lax.axis_index("s")
        for r in range(21):
            for i in range(8):
                zb_v[r, pl.ds(16 * i, 16)] = jnp.zeros((16,), jnp.float32)
        pltpu.sync_copy(gidx_hbm.at[pl.ds(s * _RPC, _RPC)],
                        idx_v.at[pl.ds(0, _RPC)])

        def run_phase(p):
            zds = [pltpu.async_copy(
                zb_v, acc.at[pl.ds(s * _NZ2 + 21 * r, 21)], zsems.at[r])
                for r in range(8)]
            pltpu.sync_copy(sidx_hbm.at[p, c, pl.ds(s * _RPC, _RPC)],
                            idx_v.at[pl.ds(_RPC, _RPC)])
            for d in zds:
                d.wait()
            plsc.subcore_barrier()

            def body(g, _):
                gds = []
                for k in range(4):
                    j = g * 4 + k
                    gds.append(pltpu.async_copy(
                        table_hbm.at[idx_v.at[j]], buf_v.at[k],
                        gsems.at[k]))
                sds = []
                for k in range(4):
                    j = g * 4 + k
                    gds[k].wait()
                    sds.append(pltpu.async_copy(
                        buf_v.at[k], acc.at[idx_v.at[_RPC + j]],
                        ssems.at[k], add=True))
                for d in sds:
                    d.wait()
                return ()
            lax.fori_loop(0, _RPC // 4, body, ())
            plsc.subcore_barrier()
            for i, (off, sz) in enumerate(((0, 128), (128, 40))):
                pltpu.sync_copy(acc.at[pl.ds(s * _NZ2 + off, sz)],
                                buf_v.at[i, pl.ds(0, sz)])
                pltpu.sync_copy(buf_v.at[i, pl.ds(0, sz)],
                                out_hbm.at[p, c, pl.ds(s * _NZ2 + off, sz)])

        run_phase(0)
        run_phase(1)

    return _sc_segsum


@functools.lru_cache(maxsize=None)
def _sc_pair_gather_k():
    @functools.partial(
        pl.kernel,
        out_type=jax.ShapeDtypeStruct((4 * _P_PAD, 128), jnp.float32),
        mesh=_sc_mesh(),
        scratch_types=[
            pltpu.VMEM((_RPT_P, 128), jnp.int32),
            pltpu.VMEM((4, 128, 128), jnp.float32),
            pltpu.SemaphoreType.DMA((4,)),
            pltpu.SemaphoreType.DMA((4,)),
        ],
    )
    def _sc_pair_gather(table_hbm, pidx_hbm, out_hbm, idx_v, buf_v, gsems,
                        wsems):
        c = lax.axis_index("c")
        s = lax.axis_index("s")
        wid = s * _NC + c
        pltpu.sync_copy(pidx_hbm.at[pl.ds(wid * _RPT_P, _RPT_P)], idx_v)

        def body(g, _):
            gds = []
            for k in range(4):
                j = g * 4 + k
                gds.append(pltpu.async_copy(
                    table_hbm.at[idx_v.at[j]], buf_v.at[k], gsems.at[k]))
            wds = []
            for k in range(4):
                j = g * 4 + k
                gds[k].wait()
                row = (wid * _RPT_P + j) * 128
                wds.append(pltpu.async_copy(
                    buf_v.at[k], out_hbm.at[pl.ds(row, 128)], wsems.at[k]))
            for d in wds:
                d.wait()
            return ()
        lax.fori_loop(0, _RPT_P // 4, body, ())

    return _sc_pair_gather


def _full_spec(ndim):
    return pl.BlockSpec(index_map=lambda i: (0,) * ndim)


def _prep_body(dego_ref, degi_ref, x_ref, xs_ref, nout_ref, nin_ref):
    dego = dego_ref[0] + dego_ref[1]
    degi = degi_ref[0] + degi_ref[1]
    nout = jnp.where(dego > 0, jax.lax.rsqrt(jnp.maximum(dego, 1e-30)), 0.0)
    nin = jnp.where(degi > 0, jax.lax.rsqrt(jnp.maximum(degi, 1e-30)), 0.0)
    xs_ref[...] = x_ref[...] * nout
    nout_ref[...] = nout
    nin_ref[...] = nin


def _prep(deg_out_p, deg_in_p, x_pad):
    return pl.pallas_call(
        _prep_body,
        out_shape=[
            jax.ShapeDtypeStruct((_N_PAD, 128), jnp.float32),
            jax.ShapeDtypeStruct((_N_PAD, 1), jnp.float32),
            jax.ShapeDtypeStruct((_N_PAD, 1), jnp.float32),
        ],
    )(deg_out_p, deg_in_p, x_pad)


def _layer0_body(agg_ref, nin_ref, nout_ref, W_ref, b_ref, out_ref):
    agg = jnp.concatenate(
        [agg_ref[0, 0, :_NQ], agg_ref[0, 1, :_NQ],
         agg_ref[1, 0, :_NQ], agg_ref[1, 1, :_NQ]], axis=0)
    h = jnp.dot(agg * nin_ref[...], W_ref[...],
                preferred_element_type=jnp.float32) + b_ref[...]
    out_ref[...] = jnp.maximum(h, 0.0) * nout_ref[...]


def _layer0(agg_h, nin, nout, W, b):
    return pl.pallas_call(
        _layer0_body,
        out_shape=jax.ShapeDtypeStruct((_N_PAD, 128), jnp.float32),
    )(agg_h, nin, nout, W, b.reshape(1, 128))


def _layer1_body(agg_ref, nin_ref, W_ref, b_ref, out_ref):
    agg = jnp.concatenate(
        [agg_ref[0, 0, :_NQ], agg_ref[0, 1, :_NQ],
         agg_ref[1, 0, :_NQ], agg_ref[1, 1, :_NQ]], axis=0)
    out_ref[...] = jnp.dot(agg * nin_ref[...], W_ref[...],
                           preferred_element_type=jnp.float32) + b_ref[...]


def _layer1(agg_h, nin, W, b):
    return pl.pallas_call(
        _layer1_body,
        out_shape=jax.ShapeDtypeStruct((_N_PAD, 128), jnp.float32),
    )(agg_h, nin, W, b.reshape(1, 128))


def _branch_body(temp_ref, hu_ref, hv_ref, noise_ref, unif_ref,
                 Wm_ref, bm_ref, Wl_ref, bl_ref, Wq_ref, bq_ref,
                 Wd1_ref, bd1_ref, WdX_ref, bdX_ref, Wa_ref, ba_ref,
                 A_ref, X_ref, mean_ref, logstd_ref, q_ref):
    npemb = hu_ref[0] + hv_ref[0]
    mean = jnp.dot(npemb, Wm_ref[...],
                   preferred_element_type=jnp.float32) + bm_ref[...]
    logstd = jnp.dot(npemb, Wl_ref[...],
                     preferred_element_type=jnp.float32) + bl_ref[...]
    q = jnp.dot(npemb, Wq_ref[...],
                preferred_element_type=jnp.float32) + bq_ref[...]
    Nz = noise_ref[...] * jnp.exp(logstd) + mean
    g = -jnp.log(-jnp.log(unif_ref[...] + _EPS) + _EPS)
    t = temp_ref[0, 0]
    Z = jax.nn.softmax((q + g) / t, axis=-1)
    M = jnp.zeros((_R_BLK, 32), jnp.float32)
    for c in range(_CAT):
        M = M + Z[:, c:c + 1] * Nz[:, c * 32:(c + 1) * 32]
    A_ref[...] = jax.nn.softmax(
        jnp.dot(M, Wa_ref[...], preferred_element_type=jnp.float32)
        + ba_ref[...], axis=-1)
    Xh = jnp.maximum(
        jnp.dot(M, Wd1_ref[...], preferred_element_type=jnp.float32)
        + bd1_ref[...], 0.0)
    X_ref[...] = jnp.maximum(
        jnp.dot(Xh, WdX_ref[...], preferred_element_type=jnp.float32)
        + bdX_ref[...], 0.0)
    mean_ref[...] = mean
    logstd_ref[...] = logstd
    q_ref[...] = q


def _branch(temp_arr, gath, ub, vb, noise, unif,
            Wm, bm, Wl, bl, Wq, bq, Wd1, bd1, WdX, bdX, Wa, ba):
    nblk = _EP // _R_BLK
    row = lambda i: (i, 0)
    return pl.pallas_call(
        _branch_body,
        grid=(nblk,),
        in_specs=[
            _full_spec(2),
            pl.BlockSpec((1, _R_BLK, 128), lambda i: (ub, i, 0)),
            pl.BlockSpec((1, _R_BLK, 128), lambda i: (vb, i, 0)),
            pl.BlockSpec((_R_BLK, _H2), row),
            pl.BlockSpec((_R_BLK, _CAT), row),
            _full_spec(2), _full_spec(2),
            _full_spec(2), _full_spec(2),
            _full_spec(2), _full_spec(2),
            _full_spec(2), _full_spec(2),
            _full_spec(2), _full_spec(2),
            _full_spec(2), _full_spec(2),
        ],
        out_specs=[
            pl.BlockSpec((_R_BLK, _CAT), row),
            pl.BlockSpec((_R_BLK, 128), row),
            pl.BlockSpec((_R_BLK, _H2), row),
            pl.BlockSpec((_R_BLK, _H2), row),
            pl.BlockSpec((_R_BLK, _CAT), row),
        ],
        out_shape=[
            jax.ShapeDtypeStruct((_EP, _CAT), jnp.float32),
            jax.ShapeDtypeStruct((_EP, 128), jnp.float32),
            jax.ShapeDtypeStruct((_EP, _H2), jnp.float32),
            jax.ShapeDtypeStruct((_EP, _H2), jnp.float32),
            jax.ShapeDtypeStruct((_EP, _CAT), jnp.float32),
        ],
    )(temp_arr, gath, gath, noise, unif,
      Wm, bm.reshape(1, _H2), Wl, bl.reshape(1, _H2),
      Wq, bq.reshape(1, _CAT), Wd1, bd1.reshape(1, 64),
      WdX, bdX.reshape(1, 128), Wa, ba.reshape(1, _CAT))


def kernel(x, edge_index, pos_edge_index, neg_edge_index, temp,
           W0, b0, W1, b1, Wm, bm, Wl, bl, Wq, bq,
           Wd1, bd1, WdX, bdX, Wa, ba):
    pad_e = _N + (jnp.arange(_E_PAD - _E, dtype=jnp.int32) % 32)
    edges_p = jnp.concatenate(
        [edge_index.astype(jnp.int32), jnp.stack([pad_e, pad_e])],
        axis=1).reshape(2, _EROWS, 128)
    srcp, dstp = edges_p[0], edges_p[1]
    dummy = _NQ + (dstp % 32)
    sidx = jnp.stack([
        jnp.where((dstp >= q * _NQ) & (dstp < (q + 1) * _NQ),
                  dstp - q * _NQ, dummy)
        for q in range(4)
    ]).reshape(2, 2, _EROWS, 128)
    pad_p = jnp.arange(_P_PAD - _EP, dtype=jnp.int32) % 32
    pidx = jnp.concatenate([
        pos_edge_index[0], pad_p, pos_edge_index[1], pad_p,
        neg_edge_index[0], pad_p, neg_edge_index[1], pad_p,
    ]).astype(jnp.int32).reshape(_PROWS, 128)
    x_pad = jnp.pad(x, ((0, _N_PAD - _N), (0, 0)))

    degs = _sc_degrees_k()(edges_p)
    deg_out_p = degs[0][..., None]
    deg_in_p = degs[1][..., None]

    xs, nout, nin = _prep(deg_out_p, deg_in_p, x_pad)
    agg0_h = _sc_segsum_k()(xs, srcp, sidx)
    h0s = _layer0(agg0_h, nin, nout, W0, b0)
    agg1_h = _sc_segsum_k()(h0s, srcp, sidx)
    h = _layer1(agg1_h, nin, W1, b1)

    gath = _sc_pair_gather_k()(h, pidx).reshape(4, _P_PAD, 128)

    temp_arr = jnp.asarray(temp, jnp.float32).reshape(1, 1)
    noise_p = jax.random.normal(jax.random.key(42), (_EP, _H2), jnp.float32)
    unif_p = jax.random.uniform(jax.random.key(43), (_EP, _CAT), jnp.float32)
    noise_n = jax.random.normal(jax.random.key(44), (_EP, _H2), jnp.float32)
    unif_n = jax.random.uniform(jax.random.key(45), (_EP, _CAT), jnp.float32)

    posA, posX, pos_mean, pos_logstd, posq = _branch(
        temp_arr, gath, 0, 1, noise_p, unif_p,
        Wm, bm, Wl, bl, Wq, bq, Wd1, bd1, WdX, bdX, Wa, ba)
    negA, negX, neg_mean, neg_logstd, negq = _branch(
        temp_arr, gath, 2, 3, noise_n, unif_n,
        Wm, bm, Wl, bl, Wq, bq, Wd1, bd1, WdX, bdX, Wa, ba)

    return (posA, negA, posX, negX, pos_mean, neg_mean,
            pos_logstd, neg_logstd, posq, negq)


# speedup vs baseline: 1.7844x; 1.1295x over previous
import functools

import jax
import jax.numpy as jnp
import numpy as np
from jax import lax
from jax.experimental import pallas as pl
from jax.experimental.pallas import tpu as pltpu
from jax.experimental.pallas import tpu_sc as plsc

_N = 10000
_E = 320000
_EP = 100000
_CAT = 8
_H2 = 256
_EPS = 1e-07
_N_PAD = 10240
_R_BLK = 2000

_NC = 2
_NS = 16
_NW = _NC * _NS
_E_PAD = 327680
_EROWS = _E_PAD // 128
_RPT = _EROWS // _NW
_P_PAD = 106496
_PROWS = 4 * _P_PAD // 128
_RPT_P = _PROWS // _NW
_NZ = _N_PAD // _NS


def _sc_mesh():
    return plsc.VectorSubcoreMesh(core_axis_name="c", subcore_axis_name="s")


@functools.lru_cache(maxsize=None)
def _sc_degrees_k():
    @functools.partial(
        pl.kernel,
        out_type=jax.ShapeDtypeStruct((2, 2, _N_PAD), jnp.float32),
        mesh=_sc_mesh(),
        scratch_types=[
            pltpu.VMEM((2 * _RPT, 128), jnp.int32),
            pltpu.VMEM((128,), jnp.float32),
            pltpu.VMEM((_NZ,), jnp.float32),
            pltpu.VMEM_SHARED((_N_PAD,), jnp.float32),
            pltpu.VMEM_SHARED((_N_PAD,), jnp.float32),
            pltpu.SemaphoreType.DMA((4,)),
        ],
    )
    def _sc_degrees(edges_hbm, out_hbm, idx_v, ones_v, zb_v, acc0, acc1,
                    sems):
        c = lax.axis_index("c")
        s = lax.axis_index("s")
        wid = s * _NC + c
        for i in range(8):
            ones_v[pl.ds(16 * i, 16)] = jnp.ones((16,), jnp.float32)
        for i in range(_NZ // 16):
            zb_v[pl.ds(16 * i, 16)] = jnp.zeros((16,), jnp.float32)
        pltpu.sync_copy(zb_v, acc0.at[pl.ds(s * _NZ, _NZ)])
        pltpu.sync_copy(zb_v, acc1.at[pl.ds(s * _NZ, _NZ)])
        for t in range(2):
            pltpu.sync_copy(edges_hbm.at[t, pl.ds(wid * _RPT, _RPT)],
                            idx_v.at[pl.ds(t * _RPT, _RPT)])
        plsc.subcore_barrier()
        for t, acc in ((0, acc0), (1, acc1)):
            def body(g, _, t=t, acc=acc):
                descs = []
                for k in range(4):
                    j = t * _RPT + g * 4 + k
                    descs.append(pltpu.async_copy(
                        ones_v, acc.at[idx_v.at[j]], sems.at[k], add=True))
                for d in descs:
                    d.wait()
                return ()
            lax.fori_loop(0, _RPT // 4, body, ())
        plsc.subcore_barrier()
        pltpu.sync_copy(acc0.at[pl.ds(s * _NZ, _NZ)],
                        out_hbm.at[0, c, pl.ds(s * _NZ, _NZ)])
        pltpu.sync_copy(acc1.at[pl.ds(s * _NZ, _NZ)],
                        out_hbm.at[1, c, pl.ds(s * _NZ, _NZ)])

    return _sc_degrees


_RPC = _EROWS // _NS
_NQ = _N_PAD // 4
_A_PAD = _NQ + 128
_NZ2 = _A_PAD // _NS


@functools.lru_cache(maxsize=None)
def _sc_segsum_k():
    @functools.partial(
        pl.kernel,
        out_type=jax.ShapeDtypeStruct((2, 2, _A_PAD, 128), jnp.float32),
        mesh=_sc_mesh(),
        scratch_types=[
            pltpu.VMEM((2 * _RPC, 128), jnp.int32),
            pltpu.VMEM((4, 128, 128), jnp.float32),
            pltpu.VMEM((21, 128), jnp.float32),
            pltpu.VMEM_SHARED((_A_PAD, 128), jnp.float32),
            pltpu.SemaphoreType.DMA((4,)),
            pltpu.SemaphoreType.DMA((4,)),
            pltpu.SemaphoreType.DMA((8,)),
        ],
    )
    def _sc_segsum(table_hbm, gidx_hbm, sidx_hbm, out_hbm,
                   idx_v, buf_v, zb_v, acc, gsems, ssems, zsems):
        c = lax.axis_index("c")
        s = lax.axis_index("s")
        for r in range(21):
            for i in range(8):
                zb_v[r, pl.ds(16 * i, 16)] = jnp.zeros((16,), jnp.float32)
        pltpu.sync_copy(gidx_hbm.at[pl.ds(s * _RPC, _RPC)],
                        idx_v.at[pl.ds(0, _RPC)])

        def run_phase(p):
            zds = [pltpu.async_copy(
                zb_v, acc.at[pl.ds(s * _NZ2 + 21 * r, 21)], zsems.at[r])
                for r in range(8)]
            pltpu.sync_copy(sidx_hbm.at[p, c, pl.ds(s * _RPC, _RPC)],
                            idx_v.at[pl.ds(_RPC, _RPC)])
            for d in zds:
                d.wait()
            plsc.subcore_barrier()

            def body(g, _):
                gds = []
                for k in range(4):
                    j = g * 4 + k
                    gds.append(pltpu.async_copy(
                        table_hbm.at[idx_v.at[j]], buf_v.at[k],
                        gsems.at[k]))
                sds = []
                for k in range(4):
                    j = g * 4 + k
                    gds[k].wait()
                    sds.append(pltpu.async_copy(
                        buf_v.at[k], acc.at[idx_v.at[_RPC + j]],
                        ssems.at[k], add=True))
                for d in sds:
                    d.wait()
                return ()
            lax.fori_loop(0, _RPC // 4, body, ())
            plsc.subcore_barrier()
            for i, (off, sz) in enumerate(((0, 128), (128, 40))):
                pltpu.sync_copy(acc.at[pl.ds(s * _NZ2 + off, sz)],
                                buf_v.at[i, pl.ds(0, sz)])
                pltpu.sync_copy(buf_v.at[i, pl.ds(0, sz)],
                                out_hbm.at[p, c, pl.ds(s * _NZ2 + off, sz)])

        run_phase(0)
        run_phase(1)

    return _sc_segsum


@functools.lru_cache(maxsize=None)
def _sc_pair_gather_k():
    @functools.partial(
        pl.kernel,
        out_type=jax.ShapeDtypeStruct((4 * _P_PAD, 128), jnp.float32),
        mesh=_sc_mesh(),
        scratch_types=[
            pltpu.VMEM((_RPT_P, 128), jnp.int32),
            pltpu.VMEM((4, 128, 128), jnp.float32),
            pltpu.SemaphoreType.DMA((4,)),
            pltpu.SemaphoreType.DMA((4,)),
        ],
    )
    def _sc_pair_gather(table_hbm, pidx_hbm, out_hbm, idx_v, buf_v, gsems,
                        wsems):
        c = lax.axis_index("c")
        s = lax.axis_index("s")
        wid = s * _NC + c
        pltpu.sync_copy(pidx_hbm.at[pl.ds(wid * _RPT_P, _RPT_P)], idx_v)

        def body(g, _):
            gds = []
            for k in range(4):
                j = g * 4 + k
                gds.append(pltpu.async_copy(
                    table_hbm.at[idx_v.at[j]], buf_v.at[k], gsems.at[k]))
            wds = []
            for k in range(4):
                j = g * 4 + k
                gds[k].wait()
                row = (wid * _RPT_P + j) * 128
                wds.append(pltpu.async_copy(
                    buf_v.at[k], out_hbm.at[pl.ds(row, 128)], wsems.at[k]))
            for d in wds:
                d.wait()
            return ()
        lax.fori_loop(0, _RPT_P // 4, body, ())

    return _sc_pair_gather


def _full_spec(ndim):
    return pl.BlockSpec(index_map=lambda i: (0,) * ndim)


def _prep_body(dego_ref, degi_ref, x_ref, xs_ref, nout_ref, nin_ref):
    dego = dego_ref[0] + dego_ref[1]
    degi = degi_ref[0] + degi_ref[1]
    nout = jnp.where(dego > 0, jax.lax.rsqrt(jnp.maximum(dego, 1e-30)), 0.0)
    nin = jnp.where(degi > 0, jax.lax.rsqrt(jnp.maximum(degi, 1e-30)), 0.0)
    xs_ref[...] = x_ref[...] * nout
    nout_ref[...] = nout
    nin_ref[...] = nin


def _prep(deg_out_p, deg_in_p, x_pad):
    return pl.pallas_call(
        _prep_body,
        out_shape=[
            jax.ShapeDtypeStruct((_N_PAD, 128), jnp.float32),
            jax.ShapeDtypeStruct((_N_PAD, 1), jnp.float32),
            jax.ShapeDtypeStruct((_N_PAD, 1), jnp.float32),
        ],
    )(deg_out_p, deg_in_p, x_pad)


def _layer0_body(agg_ref, nin_ref, nout_ref, W_ref, b_ref, out_ref):
    agg = jnp.concatenate(
        [agg_ref[0, 0, :_NQ], agg_ref[0, 1, :_NQ],
         agg_ref[1, 0, :_NQ], agg_ref[1, 1, :_NQ]], axis=0)
    h = jnp.dot(agg * nin_ref[...], W_ref[...],
                preferred_element_type=jnp.float32) + b_ref[...]
    out_ref[...] = jnp.maximum(h, 0.0) * nout_ref[...]


def _layer0(agg_h, nin, nout, W, b):
    return pl.pallas_call(
        _layer0_body,
        out_shape=jax.ShapeDtypeStruct((_N_PAD, 128), jnp.float32),
    )(agg_h, nin, nout, W, b.reshape(1, 128))


def _layer1_body(agg_ref, nin_ref, W_ref, b_ref, out_ref):
    agg = jnp.concatenate(
        [agg_ref[0, 0, :_NQ], agg_ref[0, 1, :_NQ],
         agg_ref[1, 0, :_NQ], agg_ref[1, 1, :_NQ]], axis=0)
    out_ref[...] = jnp.dot(agg * nin_ref[...], W_ref[...],
                           preferred_element_type=jnp.float32) + b_ref[...]


def _layer1(agg_h, nin, W, b):
    return pl.pallas_call(
        _layer1_body,
        out_shape=jax.ShapeDtypeStruct((_N_PAD, 128), jnp.float32),
    )(agg_h, nin, W, b.reshape(1, 128))


_NRM_LO = float(np.nextafter(np.float32(-1.0), np.float32(0.0)))
_NRM_D = float(np.float32(1.0) - np.float32(_NRM_LO))
_SQRT2 = float(np.float32(np.sqrt(2.0)))


def _threefry_bits(seed, cnt):
    ks0 = np.uint32(0)
    ks1 = np.uint32(seed)
    ks2 = np.uint32(ks0 ^ ks1 ^ np.uint32(0x1BD11BDA))
    x0 = cnt * np.uint32(0) + ks0
    x1 = cnt + ks1
    rots = ((13, 15, 26, 6), (17, 29, 16, 24))
    adds = ((ks1, ks2, 1), (ks2, ks0, 2), (ks0, ks1, 3),
            (ks1, ks2, 4), (ks2, ks0, 5))
    for g in range(5):
        for r in rots[g % 2]:
            x0 = x0 + x1
            x1 = (x1 << np.uint32(r)) | (x1 >> np.uint32(32 - r))
            x1 = x1 ^ x0
        a0, a1, inc = adds[g]
        x0 = x0 + a0
        x1 = x1 + np.uint32(a1 + np.uint32(inc))
    return x0 ^ x1


def _bits_to_unit(bits):
    fb = (bits >> np.uint32(9)) | np.uint32(0x3F800000)
    return lax.bitcast_convert_type(fb, jnp.float32) - 1.0


def _block_cnt(i, rows, cols):
    r = lax.broadcasted_iota(jnp.int32, (rows, cols), 0)
    c = lax.broadcasted_iota(jnp.int32, (rows, cols), 1)
    return ((i * rows + r) * cols + c).astype(jnp.uint32)


def _make_branch_body(nseed, useed):
  def _branch_body(temp_ref, hu_ref, hv_ref,
                   Wm_ref, bm_ref, Wl_ref, bl_ref, Wq_ref, bq_ref,
                   Wd1_ref, bd1_ref, WdX_ref, bdX_ref, Wa_ref, ba_ref,
                   A_ref, X_ref, mean_ref, logstd_ref, q_ref):
    i = pl.program_id(0)
    un = _bits_to_unit(_threefry_bits(nseed, _block_cnt(i, _R_BLK, _H2)))
    noise = _SQRT2 * lax.erf_inv(
        jnp.maximum(np.float32(_NRM_LO), un * _NRM_D + _NRM_LO))
    uu = _bits_to_unit(_threefry_bits(useed, _block_cnt(i, _R_BLK, _CAT)))
    unif = jnp.maximum(uu, 0.0)
    npemb = hu_ref[0] + hv_ref[0]
    mean = jnp.dot(npemb, Wm_ref[...],
                   preferred_element_type=jnp.float32) + bm_ref[...]
    logstd = jnp.dot(npemb, Wl_ref[...],
                     preferred_element_type=jnp.float32) + bl_ref[...]
    q = jnp.dot(npemb, Wq_ref[...],
                preferred_element_type=jnp.float32) + bq_ref[...]
    Nz = noise * jnp.exp(logstd) + mean
    g = -jnp.log(-jnp.log(unif + _EPS) + _EPS)
    t = temp_ref[0, 0]
    Z = jax.nn.softmax((q + g) / t, axis=-1)
    M = jnp.zeros((_R_BLK, 32), jnp.float32)
    for c in range(_CAT):
        M = M + Z[:, c:c + 1] * Nz[:, c * 32:(c + 1) * 32]
    A_ref[...] = jax.nn.softmax(
        jnp.dot(M, Wa_ref[...], preferred_element_type=jnp.float32)
        + ba_ref[...], axis=-1)
    Xh = jnp.maximum(
        jnp.dot(M, Wd1_ref[...], preferred_element_type=jnp.float32)
        + bd1_ref[...], 0.0)
    X_ref[...] = jnp.maximum(
        jnp.dot(Xh, WdX_ref[...], preferred_element_type=jnp.float32)
        + bdX_ref[...], 0.0)
    mean_ref[...] = mean
    logstd_ref[...] = logstd
    q_ref[...] = q
  return _branch_body


def _branch(temp_arr, gath, ub, vb, nseed, useed,
            Wm, bm, Wl, bl, Wq, bq, Wd1, bd1, WdX, bdX, Wa, ba):
    nblk = _EP // _R_BLK
    row = lambda i: (i, 0)
    return pl.pallas_call(
        _make_branch_body(nseed, useed),
        grid=(nblk,),
        in_specs=[
            _full_spec(2),
            pl.BlockSpec((1, _R_BLK, 128), lambda i: (ub, i, 0)),
            pl.BlockSpec((1, _R_BLK, 128), lambda i: (vb, i, 0)),
            _full_spec(2), _full_spec(2),
            _full_spec(2), _full_spec(2),
            _full_spec(2), _full_spec(2),
            _full_spec(2), _full_spec(2),
            _full_spec(2), _full_spec(2),
            _full_spec(2), _full_spec(2),
        ],
        out_specs=[
            pl.BlockSpec((_R_BLK, _CAT), row),
            pl.BlockSpec((_R_BLK, 128), row),
            pl.BlockSpec((_R_BLK, _H2), row),
            pl.BlockSpec((_R_BLK, _H2), row),
            pl.BlockSpec((_R_BLK, _CAT), row),
        ],
        out_shape=[
            jax.ShapeDtypeStruct((_EP, _CAT), jnp.float32),
            jax.ShapeDtypeStruct((_EP, 128), jnp.float32),
            jax.ShapeDtypeStruct((_EP, _H2), jnp.float32),
            jax.ShapeDtypeStruct((_EP, _H2), jnp.float32),
            jax.ShapeDtypeStruct((_EP, _CAT), jnp.float32),
        ],
    )(temp_arr, gath, gath,
      Wm, bm.reshape(1, _H2), Wl, bl.reshape(1, _H2),
      Wq, bq.reshape(1, _CAT), Wd1, bd1.reshape(1, 64),
      WdX, bdX.reshape(1, 128), Wa, ba.reshape(1, _CAT))


def kernel(x, edge_index, pos_edge_index, neg_edge_index, temp,
           W0, b0, W1, b1, Wm, bm, Wl, bl, Wq, bq,
           Wd1, bd1, WdX, bdX, Wa, ba):
    pad_e = _N + (jnp.arange(_E_PAD - _E, dtype=jnp.int32) % 32)
    edges_p = jnp.concatenate(
        [edge_index.astype(jnp.int32), jnp.stack([pad_e, pad_e])],
        axis=1).reshape(2, _EROWS, 128)
    srcp, dstp = edges_p[0], edges_p[1]
    dummy = _NQ + (dstp % 32)
    sidx = jnp.stack([
        jnp.where((dstp >= q * _NQ) & (dstp < (q + 1) * _NQ),
                  dstp - q * _NQ, dummy)
        for q in range(4)
    ]).reshape(2, 2, _EROWS, 128)
    pad_p = jnp.arange(_P_PAD - _EP, dtype=jnp.int32) % 32
    pidx = jnp.concatenate([
        pos_edge_index[0], pad_p, pos_edge_index[1], pad_p,
        neg_edge_index[0], pad_p, neg_edge_index[1], pad_p,
    ]).astype(jnp.int32).reshape(_PROWS, 128)
    x_pad = jnp.pad(x, ((0, _N_PAD - _N), (0, 0)))

    degs = _sc_degrees_k()(edges_p)
    deg_out_p = degs[0][..., None]
    deg_in_p = degs[1][..., None]

    xs, nout, nin = _prep(deg_out_p, deg_in_p, x_pad)
    agg0_h = _sc_segsum_k()(xs, srcp, sidx)
    h0s = _layer0(agg0_h, nin, nout, W0, b0)
    agg1_h = _sc_segsum_k()(h0s, srcp, sidx)
    h = _layer1(agg1_h, nin, W1, b1)

    gath = _sc_pair_gather_k()(h, pidx).reshape(4, _P_PAD, 128)

    temp_arr = jnp.asarray(temp, jnp.float32).reshape(1, 1)

    posA, posX, pos_mean, pos_logstd, posq = _branch(
        temp_arr, gath, 0, 1, 42, 43,
        Wm, bm, Wl, bl, Wq, bq, Wd1, bd1, WdX, bdX, Wa, ba)
    negA, negX, neg_mean, neg_logstd, negq = _branch(
        temp_arr, gath, 2, 3, 44, 45,
        Wm, bm, Wl, bl, Wq, bq, Wd1, bd1, WdX, bdX, Wa, ba)

    return (posA, negA, posX, negX, pos_mean, neg_mean,
            pos_logstd, neg_logstd, posq, negq)


# speedup vs baseline: 1.8194x; 1.0196x over previous
import functools

import jax
import jax.numpy as jnp
import numpy as np
from jax import lax
from jax.experimental import pallas as pl
from jax.experimental.pallas import tpu as pltpu
from jax.experimental.pallas import tpu_sc as plsc

_N = 10000
_E = 320000
_EP = 100000
_CAT = 8
_H2 = 256
_EPS = 1e-07
_N_PAD = 10240
_R_BLK = 2000

_NC = 2
_NS = 16
_NW = _NC * _NS
_E_PAD = 327680
_EROWS = _E_PAD // 128
_RPT = _EROWS // _NW
_P_PAD = 114688
_PROWS = 2 * _P_PAD // 128
_RPT_P = _PROWS // _NW
_NZ = _N_PAD // _NS


def _sc_mesh():
    return plsc.VectorSubcoreMesh(core_axis_name="c", subcore_axis_name="s")


@functools.lru_cache(maxsize=None)
def _sc_degrees_k():
    @functools.partial(
        pl.kernel,
        out_type=jax.ShapeDtypeStruct((2, 2, _N_PAD), jnp.float32),
        mesh=_sc_mesh(),
        scratch_types=[
            pltpu.VMEM((2 * _RPT, 128), jnp.int32),
            pltpu.VMEM((128,), jnp.float32),
            pltpu.VMEM((_NZ,), jnp.float32),
            pltpu.VMEM_SHARED((_N_PAD,), jnp.float32),
            pltpu.VMEM_SHARED((_N_PAD,), jnp.float32),
            pltpu.SemaphoreType.DMA((4,)),
        ],
    )
    def _sc_degrees(edges_hbm, out_hbm, idx_v, ones_v, zb_v, acc0, acc1,
                    sems):
        c = lax.axis_index("c")
        s = lax.axis_index("s")
        wid = s * _NC + c
        for i in range(8):
            ones_v[pl.ds(16 * i, 16)] = jnp.ones((16,), jnp.float32)
        for i in range(_NZ // 16):
            zb_v[pl.ds(16 * i, 16)] = jnp.zeros((16,), jnp.float32)
        pltpu.sync_copy(zb_v, acc0.at[pl.ds(s * _NZ, _NZ)])
        pltpu.sync_copy(zb_v, acc1.at[pl.ds(s * _NZ, _NZ)])
        for t in range(2):
            pltpu.sync_copy(edges_hbm.at[t, pl.ds(wid * _RPT, _RPT)],
                            idx_v.at[pl.ds(t * _RPT, _RPT)])
        plsc.subcore_barrier()
        for t, acc in ((0, acc0), (1, acc1)):
            def body(g, _, t=t, acc=acc):
                descs = []
                for k in range(4):
                    j = t * _RPT + g * 4 + k
                    descs.append(pltpu.async_copy(
                        ones_v, acc.at[idx_v.at[j]], sems.at[k], add=True))
                for d in descs:
                    d.wait()
                return ()
            lax.fori_loop(0, _RPT // 4, body, ())
        plsc.subcore_barrier()
        pltpu.sync_copy(acc0.at[pl.ds(s * _NZ, _NZ)],
                        out_hbm.at[0, c, pl.ds(s * _NZ, _NZ)])
        pltpu.sync_copy(acc1.at[pl.ds(s * _NZ, _NZ)],
                        out_hbm.at[1, c, pl.ds(s * _NZ, _NZ)])

    return _sc_degrees


_RPC = _EROWS // _NS
_NQ = _N_PAD // 4
_A_PAD = _NQ + 128
_NZ2 = _A_PAD // _NS


@functools.lru_cache(maxsize=None)
def _sc_segsum_k():
    @functools.partial(
        pl.kernel,
        out_type=jax.ShapeDtypeStruct((2, 2, _A_PAD, 128), jnp.float32),
        mesh=_sc_mesh(),
        scratch_types=[
            pltpu.VMEM((2 * _RPC, 128), jnp.int32),
            pltpu.VMEM((4, 128, 128), jnp.float32),
            pltpu.VMEM((21, 128), jnp.float32),
            pltpu.VMEM_SHARED((_A_PAD, 128), jnp.float32),
            pltpu.SemaphoreType.DMA((4,)),
            pltpu.SemaphoreType.DMA((4,)),
            pltpu.SemaphoreType.DMA((8,)),
        ],
    )
    def _sc_segsum(table_hbm, gidx_hbm, sidx_hbm, out_hbm,
                   idx_v, buf_v, zb_v, acc, gsems, ssems, zsems):
        c = lax.axis_index("c")
        s = lax.axis_index("s")
        for r in range(21):
            for i in range(8):
                zb_v[r, pl.ds(16 * i, 16)] = jnp.zeros((16,), jnp.float32)
        pltpu.sync_copy(gidx_hbm.at[pl.ds(s * _RPC, _RPC)],
                        idx_v.at[pl.ds(0, _RPC)])

        def run_phase(p):
            zds = [pltpu.async_copy(
                zb_v, acc.at[pl.ds(s * _NZ2 + 21 * r, 21)], zsems.at[r])
                for r in range(8)]
            pltpu.sync_copy(sidx_hbm.at[p, c, pl.ds(s * _RPC, _RPC)],
                            idx_v.at[pl.ds(_RPC, _RPC)])
            for d in zds:
                d.wait()
            plsc.subcore_barrier()

            def body(g, _):
                gds = []
                for k in range(4):
                    j = g * 4 + k
                    gds.append(pltpu.async_copy(
                        table_hbm.at[idx_v.at[j]], buf_v.at[k],
                        gsems.at[k]))
                sds = []
                for k in range(4):
                    j = g * 4 + k
                    gds[k].wait()
                    sds.append(pltpu.async_copy(
                        buf_v.at[k], acc.at[idx_v.at[_RPC + j]],
                        ssems.at[k], add=True))
                for d in sds:
                    d.wait()
                return ()
            lax.fori_loop(0, _RPC // 4, body, ())
            plsc.subcore_barrier()
            for i, (off, sz) in enumerate(((0, 128), (128, 40))):
                pltpu.sync_copy(acc.at[pl.ds(s * _NZ2 + off, sz)],
                                buf_v.at[i, pl.ds(0, sz)])
                pltpu.sync_copy(buf_v.at[i, pl.ds(0, sz)],
                                out_hbm.at[p, c, pl.ds(s * _NZ2 + off, sz)])

        run_phase(0)
        run_phase(1)

    return _sc_segsum


@functools.lru_cache(maxsize=None)
def _sc_pair_gather_k():
    @functools.partial(
        pl.kernel,
        out_type=jax.ShapeDtypeStruct((2 * _P_PAD, 128), jnp.float32),
        mesh=_sc_mesh(),
        scratch_types=[
            pltpu.VMEM((_RPT_P, 128), jnp.int32),
            pltpu.VMEM((4, 128, 128), jnp.float32),
            pltpu.SemaphoreType.DMA((4,)),
            pltpu.SemaphoreType.DMA((4,)),
        ],
    )
    def _sc_pair_gather(table_hbm, pidx_hbm, out_hbm, idx_v, buf_v, gsems,
                        wsems):
        c = lax.axis_index("c")
        s = lax.axis_index("s")
        wid = s * _NC + c
        pltpu.sync_copy(pidx_hbm.at[pl.ds(wid * _RPT_P, _RPT_P)], idx_v)

        def body(g, _):
            gds = []
            for k in range(4):
                j = g * 4 + k
                gds.append(pltpu.async_copy(
                    table_hbm.at[idx_v.at[j]], buf_v.at[k], gsems.at[k]))
            wds = []
            for k in range(4):
                j = g * 4 + k
                gds[k].wait()
                row = (wid * _RPT_P + j) * 128
                wds.append(pltpu.async_copy(
                    buf_v.at[k], out_hbm.at[pl.ds(row, 128)], wsems.at[k]))
            for d in wds:
                d.wait()
            return ()
        lax.fori_loop(0, _RPT_P // 4, body, ())

    return _sc_pair_gather


def _full_spec(ndim):
    return pl.BlockSpec(index_map=lambda i: (0,) * ndim)


def _prep_body(dego_ref, degi_ref, x_ref, xs_ref, nout_ref, nin_ref):
    dego = dego_ref[0] + dego_ref[1]
    degi = degi_ref[0] + degi_ref[1]
    nout = jnp.where(dego > 0, jax.lax.rsqrt(jnp.maximum(dego, 1e-30)), 0.0)
    nin = jnp.where(degi > 0, jax.lax.rsqrt(jnp.maximum(degi, 1e-30)), 0.0)
    xs_ref[...] = x_ref[...] * nout
    nout_ref[...] = nout
    nin_ref[...] = nin


def _prep(deg_out_p, deg_in_p, x_pad):
    return pl.pallas_call(
        _prep_body,
        out_shape=[
            jax.ShapeDtypeStruct((_N_PAD, 128), jnp.float32),
            jax.ShapeDtypeStruct((_N_PAD, 1), jnp.float32),
            jax.ShapeDtypeStruct((_N_PAD, 1), jnp.float32),
        ],
    )(deg_out_p, deg_in_p, x_pad)


def _layer0_body(agg_ref, nin_ref, nout_ref, W_ref, b_ref, out_ref):
    agg = jnp.concatenate(
        [agg_ref[0, 0, :_NQ], agg_ref[0, 1, :_NQ],
         agg_ref[1, 0, :_NQ], agg_ref[1, 1, :_NQ]], axis=0)
    h = jnp.dot(agg * nin_ref[...], W_ref[...],
                preferred_element_type=jnp.float32) + b_ref[...]
    out_ref[...] = jnp.maximum(h, 0.0) * nout_ref[...]


def _layer0(agg_h, nin, nout, W, b):
    return pl.pallas_call(
        _layer0_body,
        out_shape=jax.ShapeDtypeStruct((_N_PAD, 128), jnp.float32),
    )(agg_h, nin, nout, W, b.reshape(1, 128))


def _layer1_body(agg_ref, nin_ref, W_ref, b_ref, out_ref):
    agg = jnp.concatenate(
        [agg_ref[0, 0, :_NQ], agg_ref[0, 1, :_NQ],
         agg_ref[1, 0, :_NQ], agg_ref[1, 1, :_NQ]], axis=0)
    out_ref[...] = jnp.dot(agg * nin_ref[...], W_ref[...],
                           preferred_element_type=jnp.float32) + b_ref[...]


def _layer1(agg_h, nin, W, b):
    return pl.pallas_call(
        _layer1_body,
        out_shape=jax.ShapeDtypeStruct((_N_PAD, 128), jnp.float32),
    )(agg_h, nin, W, b.reshape(1, 128))


_NRM_LO = float(np.nextafter(np.float32(-1.0), np.float32(0.0)))
_NRM_D = float(np.float32(1.0) - np.float32(_NRM_LO))
_SQRT2 = float(np.float32(np.sqrt(2.0)))


def _threefry_bits(seed, cnt):
    ks0 = np.uint32(0)
    ks1 = np.uint32(seed)
    ks2 = np.uint32(ks0 ^ ks1 ^ np.uint32(0x1BD11BDA))
    x0 = cnt * np.uint32(0) + ks0
    x1 = cnt + ks1
    rots = ((13, 15, 26, 6), (17, 29, 16, 24))
    adds = ((ks1, ks2, 1), (ks2, ks0, 2), (ks0, ks1, 3),
            (ks1, ks2, 4), (ks2, ks0, 5))
    for g in range(5):
        for r in rots[g % 2]:
            x0 = x0 + x1
            x1 = (x1 << np.uint32(r)) | (x1 >> np.uint32(32 - r))
            x1 = x1 ^ x0
        a0, a1, inc = adds[g]
        x0 = x0 + a0
        x1 = x1 + np.uint32(a1 + np.uint32(inc))
    return x0 ^ x1


def _bits_to_unit(bits):
    fb = (bits >> np.uint32(9)) | np.uint32(0x3F800000)
    return lax.bitcast_convert_type(fb, jnp.float32) - 1.0


def _block_cnt(i, rows, cols):
    r = lax.broadcasted_iota(jnp.int32, (rows, cols), 0)
    c = lax.broadcasted_iota(jnp.int32, (rows, cols), 1)
    return ((i * rows + r) * cols + c).astype(jnp.uint32)


def _make_branch_body(nseed, useed):
  def _branch_body(temp_ref, hu_ref, hv_ref,
                   Wm_ref, bm_ref, Wl_ref, bl_ref, Wq_ref, bq_ref,
                   Wd1_ref, bd1_ref, WdX_ref, bdX_ref, Wa_ref, ba_ref,
                   A_ref, X_ref, mean_ref, logstd_ref, q_ref):
    i = pl.program_id(0)
    un = _bits_to_unit(_threefry_bits(nseed, _block_cnt(i, _R_BLK, _H2)))
    noise = _SQRT2 * lax.erf_inv(
        jnp.maximum(np.float32(_NRM_LO), un * _NRM_D + _NRM_LO))
    uu = _bits_to_unit(_threefry_bits(useed, _block_cnt(i, _R_BLK, _CAT)))
    unif = jnp.maximum(uu, 0.0)
    npemb = hu_ref[0] + hv_ref[0]
    mean = jnp.dot(npemb, Wm_ref[...],
                   preferred_element_type=jnp.float32) + bm_ref[...]
    logstd = jnp.dot(npemb, Wl_ref[...],
                     preferred_element_type=jnp.float32) + bl_ref[...]
    q = jnp.dot(npemb, Wq_ref[...],
                preferred_element_type=jnp.float32) + bq_ref[...]
    Nz = noise * jnp.exp(logstd) + mean
    g = -jnp.log(-jnp.log(unif + _EPS) + _EPS)
    t = temp_ref[0, 0]
    Z = jax.nn.softmax((q + g) / t, axis=-1)
    M = jnp.zeros((_R_BLK, 32), jnp.float32)
    for c in range(_CAT):
        M = M + Z[:, c:c + 1] * Nz[:, c * 32:(c + 1) * 32]
    A_ref[...] = jax.nn.softmax(
        jnp.dot(M, Wa_ref[...], preferred_element_type=jnp.float32)
        + ba_ref[...], axis=-1)
    Xh = jnp.maximum(
        jnp.dot(M, Wd1_ref[...], preferred_element_type=jnp.float32)
        + bd1_ref[...], 0.0)
    X_ref[...] = jnp.maximum(
        jnp.dot(Xh, WdX_ref[...], preferred_element_type=jnp.float32)
        + bdX_ref[...], 0.0)
    mean_ref[...] = mean
    logstd_ref[...] = logstd
    q_ref[...] = q
  return _branch_body


def _branch(temp_arr, gath, ub, vb, nseed, useed,
            Wm, bm, Wl, bl, Wq, bq, Wd1, bd1, WdX, bdX, Wa, ba):
    nblk = _EP // _R_BLK
    row = lambda i: (i, 0)
    return pl.pallas_call(
        _make_branch_body(nseed, useed),
        grid=(nblk,),
        in_specs=[
            _full_spec(2),
            pl.BlockSpec((1, _R_BLK, 128), lambda i: (ub, i, 0)),
            pl.BlockSpec((1, _R_BLK, 128), lambda i: (vb, i, 0)),
            _full_spec(2), _full_spec(2),
            _full_spec(2), _full_spec(2),
            _full_spec(2), _full_spec(2),
            _full_spec(2), _full_spec(2),
            _full_spec(2), _full_spec(2),
            _full_spec(2), _full_spec(2),
        ],
        out_specs=[
            pl.BlockSpec((_R_BLK, _CAT), row),
            pl.BlockSpec((_R_BLK, 128), row),
            pl.BlockSpec((_R_BLK, _H2), row),
            pl.BlockSpec((_R_BLK, _H2), row),
            pl.BlockSpec((_R_BLK, _CAT), row),
        ],
        out_shape=[
            jax.ShapeDtypeStruct((_EP, _CAT), jnp.float32),
            jax.ShapeDtypeStruct((_EP, 128), jnp.float32),
            jax.ShapeDtypeStruct((_EP, _H2), jnp.float32),
            jax.ShapeDtypeStruct((_EP, _H2), jnp.float32),
            jax.ShapeDtypeStruct((_EP, _CAT), jnp.float32),
        ],
    )(temp_arr, gath, gath,
      Wm, bm.reshape(1, _H2), Wl, bl.reshape(1, _H2),
      Wq, bq.reshape(1, _CAT), Wd1, bd1.reshape(1, 64),
      WdX, bdX.reshape(1, 128), Wa, ba.reshape(1, _CAT))


def kernel(x, edge_index, pos_edge_index, neg_edge_index, temp,
           W0, b0, W1, b1, Wm, bm, Wl, bl, Wq, bq,
           Wd1, bd1, WdX, bdX, Wa, ba):
    pad_e = _N + (jnp.arange(_E_PAD - _E, dtype=jnp.int32) % 32)
    edges_p = jnp.concatenate(
        [edge_index.astype(jnp.int32), jnp.stack([pad_e, pad_e])],
        axis=1).reshape(2, _EROWS, 128)
    srcp, dstp = edges_p[0], edges_p[1]
    dummy = _NQ + (dstp % 32)
    sidx = jnp.stack([
        jnp.where((dstp >= q * _NQ) & (dstp < (q + 1) * _NQ),
                  dstp - q * _NQ, dummy)
        for q in range(4)
    ]).reshape(2, 2, _EROWS, 128)
    pad_p = jnp.arange(_P_PAD - _EP, dtype=jnp.int32) % 32
    pidx_pos = jnp.concatenate([
        pos_edge_index[0], pad_p, pos_edge_index[1], pad_p,
    ]).astype(jnp.int32).reshape(_PROWS, 128)
    pidx_neg = jnp.concatenate([
        neg_edge_index[0], pad_p, neg_edge_index[1], pad_p,
    ]).astype(jnp.int32).reshape(_PROWS, 128)
    x_pad = jnp.pad(x, ((0, _N_PAD - _N), (0, 0)))

    degs = _sc_degrees_k()(edges_p)
    deg_out_p = degs[0][..., None]
    deg_in_p = degs[1][..., None]

    xs, nout, nin = _prep(deg_out_p, deg_in_p, x_pad)
    agg0_h = _sc_segsum_k()(xs, srcp, sidx)
    h0s = _layer0(agg0_h, nin, nout, W0, b0)
    agg1_h = _sc_segsum_k()(h0s, srcp, sidx)
    h = _layer1(agg1_h, nin, W1, b1)

    temp_arr = jnp.asarray(temp, jnp.float32).reshape(1, 1)

    gath_p = _sc_pair_gather_k()(h, pidx_pos).reshape(2, _P_PAD, 128)
    posA, posX, pos_mean, pos_logstd, posq = _branch(
        temp_arr, gath_p, 0, 1, 42, 43,
        Wm, bm, Wl, bl, Wq, bq, Wd1, bd1, WdX, bdX, Wa, ba)
    gath_n = _sc_pair_gather_k()(h, pidx_neg).reshape(2, _P_PAD, 128)
    negA, negX, neg_mean, neg_logstd, negq = _branch(
        temp_arr, gath_n, 0, 1, 44, 45,
        Wm, bm, Wl, bl, Wq, bq, Wd1, bd1, WdX, bdX, Wa, ba)

    return (posA, negA, posX, negX, pos_mean, neg_mean,
            pos_logstd, neg_logstd, posq, negq)
